# Initial kernel scaffold; baseline (speedup 1.0000x reference)
import jax
import jax.numpy as jnp
from jax.experimental import pallas as pl


def kernel(h_V, h_E, mask_V, mask_attend, params, E_idx):
    raise NotImplementedError("write your pallas kernel here")


# speedup vs baseline: 6.8561x; 6.8561x over previous
import functools

import jax
import jax.numpy as jnp
from jax import lax
from jax.experimental import pallas as pl
from jax.experimental.pallas import tpu as pltpu
from jax.experimental.pallas import tpu_sc as plsc

D = 128
FF = 512
B, N, K = 2, 4096, 36
SCALE = 36.0
BN = B * N
R = B * N * K
TB = 128
TE = TB * K
GRID = BN // TB

_gelu = jax.nn.gelu


def _ln(x, g, b):
    m = jnp.mean(x, -1, keepdims=True)
    v = jnp.mean((x - m) ** 2, -1, keepdims=True)
    return (x - m) / jnp.sqrt(v + 1e-5) * g + b


def _full(shape):
    return pl.BlockSpec(shape, lambda i: (0,) * len(shape))


def _tca_body(hv, w1a, b1, w1c, a1, hv3):
    x = hv[...]
    a1[...] = jnp.dot(x, w1a[...], preferred_element_type=jnp.float32) + b1[...]
    hv3[...] = jnp.dot(x, w1c[...], preferred_element_type=jnp.float32)


def _tc_a(hv, w1a, b1, w1c):
    t = 1024
    return pl.pallas_call(
        _tca_body,
        grid=(BN // t,),
        in_specs=[
            pl.BlockSpec((t, D), lambda i: (i, 0)),
            _full((D, D)), _full((1, D)), _full((D, D)),
        ],
        out_specs=[pl.BlockSpec((t, D), lambda i: (i, 0))] * 2,
        out_shape=[jax.ShapeDtypeStruct((BN, D), jnp.float32)] * 2,
    )(hv, w1a, b1, w1c)


def _tcb_body(hv, a1, he, g1, m_att, mask_v,
              w1b, w2, b2, w3, b3, win, bin_, wout, bout,
              ln1g, ln1b, ln2g, ln2b, w11a, b11, w11c,
              h_mid, a2, hv3b):
    e1 = jnp.dot(he[...], w1b[...], preferred_element_type=jnp.float32)
    x = (e1 + g1[...]).reshape(TB, K, D) + a1[...][:, None, :]
    u = _gelu(x).reshape(TE, D)
    u = _gelu(jnp.dot(u, w2[...], preferred_element_type=jnp.float32) + b2[...])
    m = m_att[...]
    u = u * m
    usum = jnp.sum(u.reshape(TB, K, D), axis=1)
    msum = jnp.sum(m.reshape(TB, K, 1), axis=1)
    dh = (jnp.dot(usum, w3[...], preferred_element_type=jnp.float32)
          + b3[...] * msum) / SCALE
    h = _ln(hv[...] + dh, ln1g[...], ln1b[...])
    f = _gelu(jnp.dot(h, win[...], preferred_element_type=jnp.float32) + bin_[...])
    dh2 = jnp.dot(f, wout[...], preferred_element_type=jnp.float32) + bout[...]
    h = _ln(h + dh2, ln2g[...], ln2b[...]) * mask_v[...]
    h_mid[...] = h
    a2[...] = jnp.dot(h, w11a[...], preferred_element_type=jnp.float32) + b11[...]
    hv3b[...] = jnp.dot(h, w11c[...], preferred_element_type=jnp.float32)


def _tc_b(hv, a1, he, g1, m_att, mask_v, p):
    return pl.pallas_call(
        _tcb_body,
        grid=(GRID,),
        in_specs=[
            pl.BlockSpec((TB, D), lambda i: (i, 0)),
            pl.BlockSpec((TB, D), lambda i: (i, 0)),
            pl.BlockSpec((TE, D), lambda i: (i, 0)),
            pl.BlockSpec((TE, D), lambda i: (i, 0)),
            pl.BlockSpec((TE, 1), lambda i: (i, 0)),
            pl.BlockSpec((TB, 1), lambda i: (i, 0)),
            _full((D, D)), _full((D, D)), _full((1, D)), _full((D, D)),
            _full((1, D)), _full((D, FF)), _full((1, FF)), _full((FF, D)),
            _full((1, D)), _full((1, D)), _full((1, D)), _full((1, D)),
            _full((1, D)), _full((D, D)), _full((1, D)), _full((D, D)),
        ],
        out_specs=[pl.BlockSpec((TB, D), lambda i: (i, 0))] * 3,
        out_shape=[jax.ShapeDtypeStruct((BN, D), jnp.float32)] * 3,
    )(hv, a1, he, g1, m_att, mask_v,
      p['w1b'], p['w2'], p['b2'], p['w3'], p['b3'],
      p['win'], p['bin'], p['wout'], p['bout'],
      p['ln1g'], p['ln1b'], p['ln2g'], p['ln2b'],
      p['w11a'], p['b11'], p['w11c'])


def _tcc_body(he, g2, a2, m_att, w11b, w12, b12, w13, b13, ln3g, ln3b,
              he_out, neigh):
    hee = he[...]
    e2 = jnp.dot(hee, w11b[...], preferred_element_type=jnp.float32)
    x = (e2 + g2[...]).reshape(TB, K, D) + a2[...][:, None, :]
    u = _gelu(x).reshape(TE, D)
    u = _gelu(jnp.dot(u, w12[...], preferred_element_type=jnp.float32) + b12[...])
    msg2 = jnp.dot(u, w13[...], preferred_element_type=jnp.float32) + b13[...]
    ho = _ln(hee + msg2, ln3g[...], ln3b[...])
    he_out[...] = ho
    neigh[...] = ho * m_att[...]


def _tc_c(he, g2, a2, m_att, p):
    return pl.pallas_call(
        _tcc_body,
        grid=(GRID,),
        in_specs=[
            pl.BlockSpec((TE, D), lambda i: (i, 0)),
            pl.BlockSpec((TE, D), lambda i: (i, 0)),
            pl.BlockSpec((TB, D), lambda i: (i, 0)),
            pl.BlockSpec((TE, 1), lambda i: (i, 0)),
            _full((D, D)), _full((D, D)), _full((1, D)), _full((D, D)),
            _full((1, D)), _full((1, D)), _full((1, D)),
        ],
        out_specs=[pl.BlockSpec((TE, D), lambda i: (i, 0))] * 2,
        out_shape=[jax.ShapeDtypeStruct((R, D), jnp.float32)] * 2,
    )(he, g2, a2, m_att,
      p['w11b'], p['w12'], p['b12'], p['w13'], p['b13'],
      p['ln3g'], p['ln3b'])


def _tcd_body(vals, cnts, h_mid, t1w, t1b, t2w, t2b,
              ln4g, ln4b, ln5g, ln5b, out):
    v = vals[...]
    c = cnts[...]
    c = jnp.where(c == 0.0, 1.0, c)
    ne = _ln(v / c, ln4g[...], ln4b[...])
    t = jax.nn.relu(jnp.dot(ne, t1w[...], preferred_element_type=jnp.float32)
                    + t1b[...])
    t = jnp.dot(t, t2w[...], preferred_element_type=jnp.float32) + t2b[...]
    ne = _ln(ne + t, ln5g[...], ln5b[...])
    out[...] = h_mid[...] + ne


def _tc_d(vals_p, cnts_p, h_mid, p):
    t = 512
    return pl.pallas_call(
        _tcd_body,
        grid=(BN // t,),
        in_specs=[
            pl.BlockSpec((t, D), lambda i: (i, 0)),
            pl.BlockSpec((t, D), lambda i: (i, 0)),
            pl.BlockSpec((t, D), lambda i: (i, 0)),
            _full((D, D)), _full((1, D)), _full((D, D)), _full((1, D)),
            _full((1, D)), _full((1, D)), _full((1, D)), _full((1, D)),
        ],
        out_specs=pl.BlockSpec((t, D), lambda i: (i, 0)),
        out_shape=jax.ShapeDtypeStruct((BN, D), jnp.float32),
    )(vals_p, cnts_p, h_mid,
      p['t1w'], p['t1b'], p['t2w'], p['t2b'],
      p['ln4g'], p['ln4b'], p['ln5g'], p['ln5b'])


NW = 32
EPT = R // NW
CH = 128
NCH = EPT // CH


def _sc_gather(table, idx2d):
    mesh = plsc.VectorSubcoreMesh(core_axis_name="c", subcore_axis_name="s",
                                  num_cores=2, num_subcores=16)

    @functools.partial(
        pl.kernel, mesh=mesh,
        out_type=jax.ShapeDtypeStruct((R, D), jnp.float32),
        scratch_types=[
            pltpu.VMEM((NCH, CH), jnp.int32),
            pltpu.VMEM((2, CH, D), jnp.float32),
            pltpu.SemaphoreType.DMA,
            pltpu.SemaphoreType.DMA,
        ],
    )
    def k(table_hbm, idx_hbm, out_hbm, idx_v, buf, gsem, wsem):
        wid = lax.axis_index("s") * 2 + lax.axis_index("c")
        base = wid * EPT
        pltpu.sync_copy(idx_hbm.at[pl.ds(wid * NCH, NCH)], idx_v)
        pltpu.async_copy(table_hbm.at[idx_v.at[0]], buf.at[0], gsem).wait()

        def body(j, _):
            slot = lax.rem(j, 2)
            nxt = lax.rem(j + 1, 2)

            @pl.when(j + 1 < NCH)
            def _():
                pltpu.async_copy(
                    table_hbm.at[idx_v.at[j + 1]], buf.at[nxt], gsem).wait()

            pltpu.async_copy(
                buf.at[slot], out_hbm.at[pl.ds(base + j * CH, CH)], wsem).wait()
            return 0

        lax.fori_loop(0, NCH, body, 0)

    return k(table, idx2d)


def _sc_scatter(neigh, idx2d, z128, ones128):
    mesh = plsc.VectorSubcoreMesh(core_axis_name="c", subcore_axis_name="s",
                                  num_cores=2, num_subcores=16)
    RPS = BN // 16
    EPT2 = R // 16
    NCH2 = EPT2 // CH

    @functools.partial(
        pl.kernel, mesh=mesh,
        out_type=[jax.ShapeDtypeStruct((BN, D), jnp.float32),
                  jax.ShapeDtypeStruct((BN, D), jnp.float32)],
        scratch_types=[
            pltpu.VMEM_SHARED((BN, D), jnp.float32),
            pltpu.VMEM((NCH2, CH), jnp.int32),
            pltpu.VMEM((CH, D), jnp.float32),
        ],
    )
    def k(neigh_hbm, idx_hbm, z128_hbm, ones_hbm,
          vals_out, cnts_out,
          acc_sh, idx_v, rowbuf):
        cc = lax.axis_index("c")
        ss = lax.axis_index("s")
        pltpu.sync_copy(z128_hbm.at[pl.ds(ss * RPS, RPS)],
                        acc_sh.at[pl.ds(ss * RPS, RPS)])
        pltpu.sync_copy(idx_hbm.at[pl.ds(ss * NCH2, NCH2)], idx_v)

        @pl.when(cc == 1)
        def _():
            pltpu.sync_copy(ones_hbm, rowbuf)

        plsc.subcore_barrier()

        @pl.when(cc == 0)
        def _():
            def body(j, _):
                pltpu.sync_copy(
                    neigh_hbm.at[pl.ds(ss * EPT2 + j * CH, CH)], rowbuf)
                pltpu.sync_copy(rowbuf, acc_sh.at[idx_v.at[j]], add=True)
                return 0
            lax.fori_loop(0, NCH2, body, 0)

        @pl.when(cc == 1)
        def _():
            def body(j, _):
                pltpu.sync_copy(rowbuf, acc_sh.at[idx_v.at[j]], add=True)
                return 0
            lax.fori_loop(0, NCH2, body, 0)

        plsc.subcore_barrier()
        for q in range(RPS // CH):
            r0 = ss * RPS + q * CH

            @pl.when(cc == 0)
            def _():
                pltpu.sync_copy(acc_sh.at[pl.ds(r0, CH)], rowbuf)
                pltpu.sync_copy(rowbuf, vals_out.at[pl.ds(r0, CH)])

            @pl.when(cc == 1)
            def _():
                pltpu.sync_copy(acc_sh.at[pl.ds(r0, CH)], rowbuf)
                pltpu.sync_copy(rowbuf, cnts_out.at[pl.ds(r0, CH)])

    return k(neigh, idx2d, z128, ones128)


def _gather_rows(table, idx2d):
    return jnp.take(table, idx2d.reshape(-1), axis=0)


def _scatter_mean_parts(neigh, idx2d):
    flat = idx2d.reshape(-1)
    vals = jnp.zeros((BN, D), jnp.float32).at[flat].add(neigh)
    cnts = jnp.zeros((BN,), jnp.float32).at[flat].add(1.0)
    return vals[None], cnts[None, :, None]


def kernel(h_V, h_E, mask_V, mask_attend, params, E_idx):
    p = dict(params)
    hv = h_V.reshape(BN, D)
    he = h_E.reshape(R, D)
    m_att = mask_attend.reshape(R, 1)
    mask_v = mask_V.reshape(BN, 1)
    flat_idx = (E_idx.astype(jnp.int32)
                + (jnp.arange(B, dtype=jnp.int32) * N)[:, None, None])
    idx2d = flat_idx.reshape(R // 128, 128)

    p['w1a'], p['w1b'], p['w1c'] = p['w1'][:D], p['w1'][D:2 * D], p['w1'][2 * D:]
    p['w11a'], p['w11b'], p['w11c'] = (p['w11'][:D], p['w11'][D:2 * D],
                                       p['w11'][2 * D:])
    for k in ('b1', 'b2', 'b3', 'b11', 'b12', 'b13', 'bin', 'bout',
              't1b', 't2b', 'ln1g', 'ln1b', 'ln2g', 'ln2b', 'ln3g', 'ln3b',
              'ln4g', 'ln4b', 'ln5g', 'ln5b'):
        p[k] = p[k].reshape(1, -1)

    z128 = jnp.zeros((BN, D), jnp.float32)
    ones128 = jnp.ones((CH, D), jnp.float32)

    a1, hv3 = _tc_a(hv, p['w1a'], p['b1'], p['w1c'])
    g1 = _sc_gather(hv3, idx2d)
    h_mid, a2, hv3b = _tc_b(hv, a1, he, g1, m_att, mask_v, p)
    g2 = _sc_gather(hv3b, idx2d)
    he_out, neigh = _tc_c(he, g2, a2, m_att, p)
    vals, cnts = _sc_scatter(neigh, idx2d, z128, ones128)
    hv_out = _tc_d(vals, cnts, h_mid, p)
    return hv_out.reshape(B, N, D), he_out.reshape(B, N, K, D)


# speedup vs baseline: 7.3149x; 1.0669x over previous
import functools

import jax
import jax.numpy as jnp
from jax import lax
from jax.experimental import pallas as pl
from jax.experimental.pallas import tpu as pltpu
from jax.experimental.pallas import tpu_sc as plsc

D = 128
FF = 512
B, N, K = 2, 4096, 36
SCALE = 36.0
BN = B * N
R = B * N * K
TB = 128
TE = TB * K
GRID = BN // TB

_gelu = jax.nn.gelu


def _ln(x, g, b):
    m = jnp.mean(x, -1, keepdims=True)
    v = jnp.mean((x - m) ** 2, -1, keepdims=True)
    return (x - m) / jnp.sqrt(v + 1e-5) * g + b


def _full(shape):
    return pl.BlockSpec(shape, lambda i: (0,) * len(shape))


def _tca_body(hv, w1a, b1, w1c, a1, hv3):
    x = hv[...]
    a1[...] = jnp.dot(x, w1a[...], preferred_element_type=jnp.float32) + b1[...]
    hv3[...] = jnp.dot(x, w1c[...], preferred_element_type=jnp.float32)


def _tc_a(hv, w1a, b1, w1c):
    t = 1024
    return pl.pallas_call(
        _tca_body,
        grid=(BN // t,),
        in_specs=[
            pl.BlockSpec((t, D), lambda i: (i, 0)),
            _full((D, D)), _full((1, D)), _full((D, D)),
        ],
        out_specs=[pl.BlockSpec((t, D), lambda i: (i, 0))] * 2,
        out_shape=[jax.ShapeDtypeStruct((BN, D), jnp.float32)] * 2,
    )(hv, w1a, b1, w1c)


def _tcb_body(hv, a1, he, g1, m_att, mask_v,
              w1b, w2, b2, w3, b3, win, bin_, wout, bout,
              ln1g, ln1b, ln2g, ln2b, w11a, b11, w11c,
              h_mid, a2, hv3b):
    e1 = jnp.dot(he[...], w1b[...], preferred_element_type=jnp.float32)
    x = (e1 + g1[...]).reshape(TB, K, D) + a1[...][:, None, :]
    u = _gelu(x).reshape(TE, D)
    u = _gelu(jnp.dot(u, w2[...], preferred_element_type=jnp.float32) + b2[...])
    m = m_att[...]
    u = u * m
    usum = jnp.sum(u.reshape(TB, K, D), axis=1)
    msum = jnp.sum(m.reshape(TB, K, 1), axis=1)
    dh = (jnp.dot(usum, w3[...], preferred_element_type=jnp.float32)
          + b3[...] * msum) / SCALE
    h = _ln(hv[...] + dh, ln1g[...], ln1b[...])
    f = _gelu(jnp.dot(h, win[...], preferred_element_type=jnp.float32) + bin_[...])
    dh2 = jnp.dot(f, wout[...], preferred_element_type=jnp.float32) + bout[...]
    h = _ln(h + dh2, ln2g[...], ln2b[...]) * mask_v[...]
    h_mid[...] = h
    a2[...] = jnp.dot(h, w11a[...], preferred_element_type=jnp.float32) + b11[...]
    hv3b[...] = jnp.dot(h, w11c[...], preferred_element_type=jnp.float32)


def _tc_b(hv, a1, he, g1, m_att, mask_v, p):
    return pl.pallas_call(
        _tcb_body,
        grid=(GRID,),
        in_specs=[
            pl.BlockSpec((TB, D), lambda i: (i, 0)),
            pl.BlockSpec((TB, D), lambda i: (i, 0)),
            pl.BlockSpec((TE, D), lambda i: (i, 0)),
            pl.BlockSpec((TE, D), lambda i: (i, 0)),
            pl.BlockSpec((TE, 1), lambda i: (i, 0)),
            pl.BlockSpec((TB, 1), lambda i: (i, 0)),
            _full((D, D)), _full((D, D)), _full((1, D)), _full((D, D)),
            _full((1, D)), _full((D, FF)), _full((1, FF)), _full((FF, D)),
            _full((1, D)), _full((1, D)), _full((1, D)), _full((1, D)),
            _full((1, D)), _full((D, D)), _full((1, D)), _full((D, D)),
        ],
        out_specs=[pl.BlockSpec((TB, D), lambda i: (i, 0))] * 3,
        out_shape=[jax.ShapeDtypeStruct((BN, D), jnp.float32)] * 3,
    )(hv, a1, he, g1, m_att, mask_v,
      p['w1b'], p['w2'], p['b2'], p['w3'], p['b3'],
      p['win'], p['bin'], p['wout'], p['bout'],
      p['ln1g'], p['ln1b'], p['ln2g'], p['ln2b'],
      p['w11a'], p['b11'], p['w11c'])


def _tcc_body(he, g2, a2, m_att, w11b, w12, b12, w13, b13, ln3g, ln3b,
              he_out, neigh):
    hee = he[...]
    e2 = jnp.dot(hee, w11b[...], preferred_element_type=jnp.float32)
    x = (e2 + g2[...]).reshape(TB, K, D) + a2[...][:, None, :]
    u = _gelu(x).reshape(TE, D)
    u = _gelu(jnp.dot(u, w12[...], preferred_element_type=jnp.float32) + b12[...])
    msg2 = jnp.dot(u, w13[...], preferred_element_type=jnp.float32) + b13[...]
    ho = _ln(hee + msg2, ln3g[...], ln3b[...])
    he_out[...] = ho
    neigh[...] = ho * m_att[...]


def _tc_c(he, g2, a2, m_att, p):
    return pl.pallas_call(
        _tcc_body,
        grid=(GRID,),
        in_specs=[
            pl.BlockSpec((TE, D), lambda i: (i, 0)),
            pl.BlockSpec((TE, D), lambda i: (i, 0)),
            pl.BlockSpec((TB, D), lambda i: (i, 0)),
            pl.BlockSpec((TE, 1), lambda i: (i, 0)),
            _full((D, D)), _full((D, D)), _full((1, D)), _full((D, D)),
            _full((1, D)), _full((1, D)), _full((1, D)),
        ],
        out_specs=[pl.BlockSpec((TE, D), lambda i: (i, 0))] * 2,
        out_shape=[jax.ShapeDtypeStruct((R, D), jnp.float32)] * 2,
    )(he, g2, a2, m_att,
      p['w11b'], p['w12'], p['b12'], p['w13'], p['b13'],
      p['ln3g'], p['ln3b'])


def _tcd_body(vals, cnts, h_mid, t1w, t1b, t2w, t2b,
              ln4g, ln4b, ln5g, ln5b, out):
    v = vals[...]
    c = cnts[...]
    c = jnp.where(c == 0.0, 1.0, c)
    ne = _ln(v / c, ln4g[...], ln4b[...])
    t = jax.nn.relu(jnp.dot(ne, t1w[...], preferred_element_type=jnp.float32)
                    + t1b[...])
    t = jnp.dot(t, t2w[...], preferred_element_type=jnp.float32) + t2b[...]
    ne = _ln(ne + t, ln5g[...], ln5b[...])
    out[...] = h_mid[...] + ne


def _tc_d(vals_p, cnts_p, h_mid, p):
    t = 512
    return pl.pallas_call(
        _tcd_body,
        grid=(BN // t,),
        in_specs=[
            pl.BlockSpec((t, D), lambda i: (i, 0)),
            pl.BlockSpec((t, D), lambda i: (i, 0)),
            pl.BlockSpec((t, D), lambda i: (i, 0)),
            _full((D, D)), _full((1, D)), _full((D, D)), _full((1, D)),
            _full((1, D)), _full((1, D)), _full((1, D)), _full((1, D)),
        ],
        out_specs=pl.BlockSpec((t, D), lambda i: (i, 0)),
        out_shape=jax.ShapeDtypeStruct((BN, D), jnp.float32),
    )(vals_p, cnts_p, h_mid,
      p['t1w'], p['t1b'], p['t2w'], p['t2b'],
      p['ln4g'], p['ln4b'], p['ln5g'], p['ln5b'])


NW = 32
EPT = R // NW
CH = 128
NCH = EPT // CH


def _sc_gather(table, idx2d):
    mesh = plsc.VectorSubcoreMesh(core_axis_name="c", subcore_axis_name="s",
                                  num_cores=2, num_subcores=16)

    @functools.partial(
        pl.kernel, mesh=mesh,
        out_type=jax.ShapeDtypeStruct((R, D), jnp.float32),
        scratch_types=[
            pltpu.VMEM((NCH, CH), jnp.int32),
            pltpu.VMEM((2, CH, D), jnp.float32),
            pltpu.SemaphoreType.DMA,
            pltpu.SemaphoreType.DMA,
        ],
    )
    def k(table_hbm, idx_hbm, out_hbm, idx_v, buf, gsem, wsem):
        wid = lax.axis_index("s") * 2 + lax.axis_index("c")
        base = wid * EPT
        pltpu.sync_copy(idx_hbm.at[pl.ds(wid * NCH, NCH)], idx_v)
        pltpu.async_copy(table_hbm.at[idx_v.at[0]], buf.at[0], gsem).wait()

        def body(j, _):
            slot = lax.rem(j, 2)
            nxt = lax.rem(j + 1, 2)
            jn = lax.min(j + 1, NCH - 1)
            h = pltpu.async_copy(table_hbm.at[idx_v.at[jn]], buf.at[nxt], gsem)
            pltpu.async_copy(
                buf.at[slot], out_hbm.at[pl.ds(base + j * CH, CH)], wsem).wait()
            h.wait()
            return 0

        lax.fori_loop(0, NCH, body, 0)

    return k(table, idx2d)


def _sc_scatter(neigh, idx2d, z128, ones128):
    mesh = plsc.VectorSubcoreMesh(core_axis_name="c", subcore_axis_name="s",
                                  num_cores=2, num_subcores=16)
    RPS = BN // 16
    EPT2 = R // 16
    NCH2 = EPT2 // CH

    @functools.partial(
        pl.kernel, mesh=mesh,
        out_type=[jax.ShapeDtypeStruct((BN, D), jnp.float32),
                  jax.ShapeDtypeStruct((BN, D), jnp.float32)],
        scratch_types=[
            pltpu.VMEM_SHARED((BN, D), jnp.float32),
            pltpu.VMEM((NCH2, CH), jnp.int32),
            pltpu.VMEM((2, CH, D), jnp.float32),
            pltpu.SemaphoreType.DMA,
        ],
    )
    def k(neigh_hbm, idx_hbm, z128_hbm, ones_hbm,
          vals_out, cnts_out,
          acc_sh, idx_v, rowbuf, lsem):
        cc = lax.axis_index("c")
        ss = lax.axis_index("s")
        pltpu.sync_copy(z128_hbm.at[pl.ds(ss * RPS, RPS)],
                        acc_sh.at[pl.ds(ss * RPS, RPS)])
        pltpu.sync_copy(idx_hbm.at[pl.ds(ss * NCH2, NCH2)], idx_v)

        @pl.when(cc == 1)
        def _():
            pltpu.sync_copy(ones_hbm, rowbuf.at[0])

        plsc.subcore_barrier()

        @pl.when(cc == 0)
        def _():
            pltpu.async_copy(neigh_hbm.at[pl.ds(ss * EPT2, CH)],
                             rowbuf.at[0], lsem).wait()

            def body(j, _):
                slot = lax.rem(j, 2)
                nxt = lax.rem(j + 1, 2)
                jn = lax.min(j + 1, NCH2 - 1)
                h = pltpu.async_copy(
                    neigh_hbm.at[pl.ds(ss * EPT2 + jn * CH, CH)],
                    rowbuf.at[nxt], lsem)
                pltpu.sync_copy(rowbuf.at[slot], acc_sh.at[idx_v.at[j]],
                                add=True)
                h.wait()
                return 0
            lax.fori_loop(0, NCH2, body, 0)

        @pl.when(cc == 1)
        def _():
            def body(j, _):
                pltpu.sync_copy(rowbuf.at[0], acc_sh.at[idx_v.at[j]],
                                add=True)
                return 0
            lax.fori_loop(0, NCH2, body, 0)

        plsc.subcore_barrier()
        for q in range(RPS // CH):
            r0 = ss * RPS + q * CH

            @pl.when(cc == 0)
            def _():
                pltpu.sync_copy(acc_sh.at[pl.ds(r0, CH)], rowbuf.at[0])
                pltpu.sync_copy(rowbuf.at[0], vals_out.at[pl.ds(r0, CH)])

            @pl.when(cc == 1)
            def _():
                pltpu.sync_copy(acc_sh.at[pl.ds(r0, CH)], rowbuf.at[0])
                pltpu.sync_copy(rowbuf.at[0], cnts_out.at[pl.ds(r0, CH)])

    return k(neigh, idx2d, z128, ones128)


def _gather_rows(table, idx2d):
    return jnp.take(table, idx2d.reshape(-1), axis=0)


def _scatter_mean_parts(neigh, idx2d):
    flat = idx2d.reshape(-1)
    vals = jnp.zeros((BN, D), jnp.float32).at[flat].add(neigh)
    cnts = jnp.zeros((BN,), jnp.float32).at[flat].add(1.0)
    return vals[None], cnts[None, :, None]


def kernel(h_V, h_E, mask_V, mask_attend, params, E_idx):
    p = dict(params)
    hv = h_V.reshape(BN, D)
    he = h_E.reshape(R, D)
    m_att = mask_attend.reshape(R, 1)
    mask_v = mask_V.reshape(BN, 1)
    flat_idx = (E_idx.astype(jnp.int32)
                + (jnp.arange(B, dtype=jnp.int32) * N)[:, None, None])
    idx2d = flat_idx.reshape(R // 128, 128)

    p['w1a'], p['w1b'], p['w1c'] = p['w1'][:D], p['w1'][D:2 * D], p['w1'][2 * D:]
    p['w11a'], p['w11b'], p['w11c'] = (p['w11'][:D], p['w11'][D:2 * D],
                                       p['w11'][2 * D:])
    for k in ('b1', 'b2', 'b3', 'b11', 'b12', 'b13', 'bin', 'bout',
              't1b', 't2b', 'ln1g', 'ln1b', 'ln2g', 'ln2b', 'ln3g', 'ln3b',
              'ln4g', 'ln4b', 'ln5g', 'ln5b'):
        p[k] = p[k].reshape(1, -1)

    z128 = jnp.zeros((BN, D), jnp.float32)
    ones128 = jnp.ones((CH, D), jnp.float32)

    a1, hv3 = _tc_a(hv, p['w1a'], p['b1'], p['w1c'])
    g1 = _sc_gather(hv3, idx2d)
    h_mid, a2, hv3b = _tc_b(hv, a1, he, g1, m_att, mask_v, p)
    g2 = _sc_gather(hv3b, idx2d)
    he_out, neigh = _tc_c(he, g2, a2, m_att, p)
    vals, cnts = _sc_scatter(neigh, idx2d, z128, ones128)
    hv_out = _tc_d(vals, cnts, h_mid, p)
    return hv_out.reshape(B, N, D), he_out.reshape(B, N, K, D)


# speedup vs baseline: 10.3145x; 1.4101x over previous
import functools

import jax
import jax.numpy as jnp
from jax import lax
from jax.experimental import pallas as pl
from jax.experimental.pallas import tpu as pltpu
from jax.experimental.pallas import tpu_sc as plsc

D = 128
FF = 512
B, N, K = 2, 4096, 36
SCALE = 36.0
BN = B * N
R = B * N * K
TB = 128
TE = TB * K
GRID = BN // TB
NPB = N // TB

_gelu = jax.nn.gelu


def _ln(x, g, b):
    m = jnp.mean(x, -1, keepdims=True)
    v = jnp.mean((x - m) ** 2, -1, keepdims=True)
    return (x - m) / jnp.sqrt(v + 1e-5) * g + b


def _full(shape):
    return pl.BlockSpec(shape, lambda i: (0,) * len(shape))


def _tca_body(hv, w1a, b1, w1c, a1, hv3):
    x = hv[...]
    a1[...] = jnp.dot(x, w1a[...], preferred_element_type=jnp.float32) + b1[...]
    hv3[...] = jnp.dot(x, w1c[...], preferred_element_type=jnp.float32)


def _tc_a(hv, w1a, b1, w1c):
    t = 1024
    return pl.pallas_call(
        _tca_body,
        grid=(BN // t,),
        in_specs=[
            pl.BlockSpec((t, D), lambda i: (i, 0)),
            _full((D, D)), _full((1, D)), _full((D, D)),
        ],
        out_specs=[pl.BlockSpec((t, D), lambda i: (i, 0))] * 2,
        out_shape=[jax.ShapeDtypeStruct((BN, D), jnp.float32)] * 2,
    )(hv, w1a, b1, w1c)


def _tcb_body(hv, a1, he, g1, m_att, mask_v,
              w1b, w2, b2, w3, b3, win, bin_, wout, bout,
              ln1g, ln1b, ln2g, ln2b, w11a, b11, w11c,
              h_mid, a2, hv3b):
    he2 = he[...].reshape(TE, D)
    e1 = jnp.dot(he2, w1b[...], preferred_element_type=jnp.float32)
    x = (e1 + g1[...]).reshape(K, TB, D) + a1[...][None, :, :]
    u = _gelu(x).reshape(TE, D)
    u = _gelu(jnp.dot(u, w2[...], preferred_element_type=jnp.float32) + b2[...])
    m = m_att[...]
    u = u * m
    usum = jnp.sum(u.reshape(K, TB, D), axis=0)
    msum = jnp.sum(m.reshape(K, TB, 1), axis=0)
    dh = (jnp.dot(usum, w3[...], preferred_element_type=jnp.float32)
          + b3[...] * msum) / SCALE
    h = _ln(hv[...] + dh, ln1g[...], ln1b[...])
    f = _gelu(jnp.dot(h, win[...], preferred_element_type=jnp.float32) + bin_[...])
    dh2 = jnp.dot(f, wout[...], preferred_element_type=jnp.float32) + bout[...]
    h = _ln(h + dh2, ln2g[...], ln2b[...]) * mask_v[...]
    h_mid[...] = h
    a2[...] = jnp.dot(h, w11a[...], preferred_element_type=jnp.float32) + b11[...]
    hv3b[...] = jnp.dot(h, w11c[...], preferred_element_type=jnp.float32)


def _tc_b(hv, a1, he, g1, m_att, mask_v, p):
    return pl.pallas_call(
        _tcb_body,
        grid=(GRID,),
        in_specs=[
            pl.BlockSpec((TB, D), lambda i: (i, 0)),
            pl.BlockSpec((TB, D), lambda i: (i, 0)),
            pl.BlockSpec((1, K, TB, D), lambda i: (i // NPB, 0, i % NPB, 0)),
            pl.BlockSpec((TE, D), lambda i: (i, 0)),
            pl.BlockSpec((TE, 1), lambda i: (i, 0)),
            pl.BlockSpec((TB, 1), lambda i: (i, 0)),
            _full((D, D)), _full((D, D)), _full((1, D)), _full((D, D)),
            _full((1, D)), _full((D, FF)), _full((1, FF)), _full((FF, D)),
            _full((1, D)), _full((1, D)), _full((1, D)), _full((1, D)),
            _full((1, D)), _full((D, D)), _full((1, D)), _full((D, D)),
        ],
        out_specs=[pl.BlockSpec((TB, D), lambda i: (i, 0))] * 3,
        out_shape=[jax.ShapeDtypeStruct((BN, D), jnp.float32)] * 3,
    )(hv, a1, he, g1, m_att, mask_v,
      p['w1b'], p['w2'], p['b2'], p['w3'], p['b3'],
      p['win'], p['bin'], p['wout'], p['bout'],
      p['ln1g'], p['ln1b'], p['ln2g'], p['ln2b'],
      p['w11a'], p['b11'], p['w11c'])


def _tcc_body(he, g2, a2, m_att, w11b, w12, b12, w13, b13, ln3g, ln3b,
              he_out, neigh):
    hee = he[...].reshape(TE, D)
    e2 = jnp.dot(hee, w11b[...], preferred_element_type=jnp.float32)
    x = (e2 + g2[...]).reshape(K, TB, D) + a2[...][None, :, :]
    u = _gelu(x).reshape(TE, D)
    u = _gelu(jnp.dot(u, w12[...], preferred_element_type=jnp.float32) + b12[...])
    msg2 = jnp.dot(u, w13[...], preferred_element_type=jnp.float32) + b13[...]
    ho = _ln(hee + msg2, ln3g[...], ln3b[...])
    he_out[...] = ho.reshape(1, K, TB, D)
    neigh[...] = ho * m_att[...]


def _tc_c(he, g2, a2, m_att, p):
    return pl.pallas_call(
        _tcc_body,
        grid=(GRID,),
        in_specs=[
            pl.BlockSpec((1, K, TB, D), lambda i: (i // NPB, 0, i % NPB, 0)),
            pl.BlockSpec((TE, D), lambda i: (i, 0)),
            pl.BlockSpec((TB, D), lambda i: (i, 0)),
            pl.BlockSpec((TE, 1), lambda i: (i, 0)),
            _full((D, D)), _full((D, D)), _full((1, D)), _full((D, D)),
            _full((1, D)), _full((1, D)), _full((1, D)),
        ],
        out_specs=[
            pl.BlockSpec((1, K, TB, D), lambda i: (i // NPB, 0, i % NPB, 0)),
            pl.BlockSpec((TE, D), lambda i: (i, 0)),
        ],
        out_shape=[jax.ShapeDtypeStruct((B, K, N, D), jnp.float32),
                   jax.ShapeDtypeStruct((R, D), jnp.float32)],
    )(he, g2, a2, m_att,
      p['w11b'], p['w12'], p['b12'], p['w13'], p['b13'],
      p['ln3g'], p['ln3b'])


def _tcd_body(vals, cnts, h_mid, t1w, t1b, t2w, t2b,
              ln4g, ln4b, ln5g, ln5b, out):
    v = vals[...]
    c = cnts[...]
    c = jnp.where(c == 0.0, 1.0, c)
    ne = _ln(v / c, ln4g[...], ln4b[...])
    t = jax.nn.relu(jnp.dot(ne, t1w[...], preferred_element_type=jnp.float32)
                    + t1b[...])
    t = jnp.dot(t, t2w[...], preferred_element_type=jnp.float32) + t2b[...]
    ne = _ln(ne + t, ln5g[...], ln5b[...])
    out[...] = h_mid[...] + ne


def _tc_d(vals_p, cnts_p, h_mid, p):
    t = 512
    return pl.pallas_call(
        _tcd_body,
        grid=(BN // t,),
        in_specs=[
            pl.BlockSpec((t, D), lambda i: (i, 0)),
            pl.BlockSpec((t, D), lambda i: (i, 0)),
            pl.BlockSpec((t, D), lambda i: (i, 0)),
            _full((D, D)), _full((1, D)), _full((D, D)), _full((1, D)),
            _full((1, D)), _full((1, D)), _full((1, D)), _full((1, D)),
        ],
        out_specs=pl.BlockSpec((t, D), lambda i: (i, 0)),
        out_shape=jax.ShapeDtypeStruct((BN, D), jnp.float32),
    )(vals_p, cnts_p, h_mid,
      p['t1w'], p['t1b'], p['t2w'], p['t2b'],
      p['ln4g'], p['ln4b'], p['ln5g'], p['ln5b'])


NW = 32
EPT = R // NW
CH = 128
NCH = EPT // CH


def _sc_gather(table, idx2d):
    mesh = plsc.VectorSubcoreMesh(core_axis_name="c", subcore_axis_name="s",
                                  num_cores=2, num_subcores=16)

    @functools.partial(
        pl.kernel, mesh=mesh,
        out_type=jax.ShapeDtypeStruct((R, D), jnp.float32),
        scratch_types=[
            pltpu.VMEM((NCH, CH), jnp.int32),
            pltpu.VMEM((2, CH, D), jnp.float32),
            pltpu.SemaphoreType.DMA,
            pltpu.SemaphoreType.DMA,
        ],
    )
    def k(table_hbm, idx_hbm, out_hbm, idx_v, buf, gsem, wsem):
        wid = lax.axis_index("s") * 2 + lax.axis_index("c")
        base = wid * EPT
        pltpu.sync_copy(idx_hbm.at[pl.ds(wid * NCH, NCH)], idx_v)
        pltpu.async_copy(table_hbm.at[idx_v.at[0]], buf.at[0], gsem).wait()

        def body(j, _):
            slot = lax.rem(j, 2)
            nxt = lax.rem(j + 1, 2)
            jn = lax.min(j + 1, NCH - 1)
            h = pltpu.async_copy(table_hbm.at[idx_v.at[jn]], buf.at[nxt], gsem)
            pltpu.async_copy(
                buf.at[slot], out_hbm.at[pl.ds(base + j * CH, CH)], wsem).wait()
            h.wait()
            return 0

        lax.fori_loop(0, NCH, body, 0)

    return k(table, idx2d)


def _sc_scatter(neigh, idx2d, z128, ones128):
    mesh = plsc.VectorSubcoreMesh(core_axis_name="c", subcore_axis_name="s",
                                  num_cores=2, num_subcores=16)
    RPS = BN // 16
    EPT2 = R // 16
    NCH2 = EPT2 // CH

    @functools.partial(
        pl.kernel, mesh=mesh,
        out_type=[jax.ShapeDtypeStruct((BN, D), jnp.float32),
                  jax.ShapeDtypeStruct((BN, D), jnp.float32)],
        scratch_types=[
            pltpu.VMEM_SHARED((BN, D), jnp.float32),
            pltpu.VMEM((NCH2, CH), jnp.int32),
            pltpu.VMEM((2, CH, D), jnp.float32),
            pltpu.SemaphoreType.DMA,
        ],
    )
    def k(neigh_hbm, idx_hbm, z128_hbm, ones_hbm,
          vals_out, cnts_out,
          acc_sh, idx_v, rowbuf, lsem):
        cc = lax.axis_index("c")
        ss = lax.axis_index("s")
        pltpu.sync_copy(z128_hbm.at[pl.ds(ss * RPS, RPS)],
                        acc_sh.at[pl.ds(ss * RPS, RPS)])
        pltpu.sync_copy(idx_hbm.at[pl.ds(ss * NCH2, NCH2)], idx_v)

        @pl.when(cc == 1)
        def _():
            pltpu.sync_copy(ones_hbm, rowbuf.at[0])

        plsc.subcore_barrier()

        @pl.when(cc == 0)
        def _():
            pltpu.async_copy(neigh_hbm.at[pl.ds(ss * EPT2, CH)],
                             rowbuf.at[0], lsem).wait()

            def body(j, _):
                slot = lax.rem(j, 2)
                nxt = lax.rem(j + 1, 2)
                jn = lax.min(j + 1, NCH2 - 1)
                h = pltpu.async_copy(
                    neigh_hbm.at[pl.ds(ss * EPT2 + jn * CH, CH)],
                    rowbuf.at[nxt], lsem)
                pltpu.sync_copy(rowbuf.at[slot], acc_sh.at[idx_v.at[j]],
                                add=True)
                h.wait()
                return 0
            lax.fori_loop(0, NCH2, body, 0)

        @pl.when(cc == 1)
        def _():
            def body(j, _):
                pltpu.sync_copy(rowbuf.at[0], acc_sh.at[idx_v.at[j]],
                                add=True)
                return 0
            lax.fori_loop(0, NCH2, body, 0)

        plsc.subcore_barrier()
        for q in range(RPS // CH):
            r0 = ss * RPS + q * CH

            @pl.when(cc == 0)
            def _():
                pltpu.sync_copy(acc_sh.at[pl.ds(r0, CH)], rowbuf.at[0])
                pltpu.sync_copy(rowbuf.at[0], vals_out.at[pl.ds(r0, CH)])

            @pl.when(cc == 1)
            def _():
                pltpu.sync_copy(acc_sh.at[pl.ds(r0, CH)], rowbuf.at[0])
                pltpu.sync_copy(rowbuf.at[0], cnts_out.at[pl.ds(r0, CH)])

    return k(neigh, idx2d, z128, ones128)


def _gather_rows(table, idx2d):
    return jnp.take(table, idx2d.reshape(-1), axis=0)


def _scatter_mean_parts(neigh, idx2d):
    flat = idx2d.reshape(-1)
    vals = jnp.zeros((BN, D), jnp.float32).at[flat].add(neigh)
    cnts = jnp.zeros((BN,), jnp.float32).at[flat].add(1.0)
    return vals[None], cnts[None, :, None]


def kernel(h_V, h_E, mask_V, mask_attend, params, E_idx):
    p = dict(params)
    hv = h_V.reshape(BN, D)
    he_t = h_E.transpose(0, 2, 1, 3)
    m_att = (mask_attend.reshape(B, NPB, TB, K).transpose(0, 1, 3, 2)
             .reshape(R, 1))
    mask_v = mask_V.reshape(BN, 1)
    flat_idx = (E_idx.astype(jnp.int32)
                + (jnp.arange(B, dtype=jnp.int32) * N)[:, None, None])
    idx2d = (flat_idx.reshape(B, NPB, TB, K).transpose(0, 1, 3, 2)
             .reshape(R // 128, 128))

    p['w1a'], p['w1b'], p['w1c'] = p['w1'][:D], p['w1'][D:2 * D], p['w1'][2 * D:]
    p['w11a'], p['w11b'], p['w11c'] = (p['w11'][:D], p['w11'][D:2 * D],
                                       p['w11'][2 * D:])
    for k in ('b1', 'b2', 'b3', 'b11', 'b12', 'b13', 'bin', 'bout',
              't1b', 't2b', 'ln1g', 'ln1b', 'ln2g', 'ln2b', 'ln3g', 'ln3b',
              'ln4g', 'ln4b', 'ln5g', 'ln5b'):
        p[k] = p[k].reshape(1, -1)

    z128 = jnp.zeros((BN, D), jnp.float32)
    ones128 = jnp.ones((CH, D), jnp.float32)

    a1, hv3 = _tc_a(hv, p['w1a'], p['b1'], p['w1c'])
    g1 = _sc_gather(hv3, idx2d)
    h_mid, a2, hv3b = _tc_b(hv, a1, he_t, g1, m_att, mask_v, p)
    g2 = _sc_gather(hv3b, idx2d)
    he_out_t, neigh = _tc_c(he_t, g2, a2, m_att, p)
    vals, cnts = _sc_scatter(neigh, idx2d, z128, ones128)
    hv_out = _tc_d(vals, cnts, h_mid, p)
    return hv_out.reshape(B, N, D), he_out_t.transpose(0, 2, 1, 3)


# speedup vs baseline: 12.5448x; 1.2162x over previous
import functools

import jax
import jax.numpy as jnp
from jax import lax
from jax.experimental import pallas as pl
from jax.experimental.pallas import tpu as pltpu
from jax.experimental.pallas import tpu_sc as plsc

D = 128
FF = 512
B, N, K = 2, 4096, 36
SCALE = 36.0
BN = B * N
R = B * N * K
TB = 128
TE = TB * K
GRID = BN // TB
NPB = N // TB

_gelu = jax.nn.gelu


def _ln(x, g, b):
    m = jnp.mean(x, -1, keepdims=True)
    v = jnp.mean((x - m) ** 2, -1, keepdims=True)
    return (x - m) / jnp.sqrt(v + 1e-5) * g + b


def _full(shape):
    return pl.BlockSpec(shape, lambda i: (0,) * len(shape))


def _tca_body(hv, w1a, b1, w1c, a1, hv3):
    x = hv[...]
    a1[...] = jnp.dot(x, w1a[...], preferred_element_type=jnp.float32) + b1[...]
    hv3[...] = jnp.dot(x, w1c[...], preferred_element_type=jnp.float32)


def _tc_a(hv, w1a, b1, w1c):
    t = 1024
    return pl.pallas_call(
        _tca_body,
        grid=(BN // t,),
        in_specs=[
            pl.BlockSpec((t, D), lambda i: (i, 0)),
            _full((D, D)), _full((1, D)), _full((D, D)),
        ],
        out_specs=[pl.BlockSpec((t, D), lambda i: (i, 0))] * 2,
        out_shape=[jax.ShapeDtypeStruct((BN, D), jnp.float32)] * 2,
    )(hv, w1a, b1, w1c)


def _tcb_body(hv, a1, he, g1, m_att, mask_v,
              w1b, w2, b2, w3, b3, win, bin_, wout, bout,
              ln1g, ln1b, ln2g, ln2b, w11a, b11, w11c,
              h_mid, a2, hv3b):
    he2 = he[...].reshape(TE, D)
    e1 = jnp.dot(he2, w1b[...], preferred_element_type=jnp.float32)
    x = (e1 + g1[...]).reshape(K, TB, D) + a1[...][None, :, :]
    u = _gelu(x).reshape(TE, D)
    u = _gelu(jnp.dot(u, w2[...], preferred_element_type=jnp.float32) + b2[...])
    m3 = m_att[...][0][:, :, None]
    u3 = u.reshape(K, TB, D) * m3
    usum = jnp.sum(u3, axis=0)
    msum = jnp.sum(m3, axis=0)
    dh = (jnp.dot(usum, w3[...], preferred_element_type=jnp.float32)
          + b3[...] * msum) / SCALE
    h = _ln(hv[...] + dh, ln1g[...], ln1b[...])
    f = _gelu(jnp.dot(h, win[...], preferred_element_type=jnp.float32) + bin_[...])
    dh2 = jnp.dot(f, wout[...], preferred_element_type=jnp.float32) + bout[...]
    h = _ln(h + dh2, ln2g[...], ln2b[...]) * mask_v[...]
    h_mid[...] = h
    a2[...] = jnp.dot(h, w11a[...], preferred_element_type=jnp.float32) + b11[...]
    hv3b[...] = jnp.dot(h, w11c[...], preferred_element_type=jnp.float32)


def _tc_b(hv, a1, he, g1, m_att, mask_v, p):
    return pl.pallas_call(
        _tcb_body,
        grid=(GRID,),
        in_specs=[
            pl.BlockSpec((TB, D), lambda i: (i, 0)),
            pl.BlockSpec((TB, D), lambda i: (i, 0)),
            pl.BlockSpec((1, K, TB, D), lambda i: (i // NPB, 0, i % NPB, 0)),
            pl.BlockSpec((TE, D), lambda i: (i, 0)),
            pl.BlockSpec((1, K, TB), lambda i: (i, 0, 0)),
            pl.BlockSpec((TB, 1), lambda i: (i, 0)),
            _full((D, D)), _full((D, D)), _full((1, D)), _full((D, D)),
            _full((1, D)), _full((D, FF)), _full((1, FF)), _full((FF, D)),
            _full((1, D)), _full((1, D)), _full((1, D)), _full((1, D)),
            _full((1, D)), _full((D, D)), _full((1, D)), _full((D, D)),
        ],
        out_specs=[pl.BlockSpec((TB, D), lambda i: (i, 0))] * 3,
        out_shape=[jax.ShapeDtypeStruct((BN, D), jnp.float32)] * 3,
    )(hv, a1, he, g1, m_att, mask_v,
      p['w1b'], p['w2'], p['b2'], p['w3'], p['b3'],
      p['win'], p['bin'], p['wout'], p['bout'],
      p['ln1g'], p['ln1b'], p['ln2g'], p['ln2b'],
      p['w11a'], p['b11'], p['w11c'])


def _tcc_body(he, g2, a2, m_att, w11b, w12, b12, w13, b13, ln3g, ln3b,
              he_out, neigh):
    hee = he[...].reshape(TE, D)
    e2 = jnp.dot(hee, w11b[...], preferred_element_type=jnp.float32)
    x = (e2 + g2[...]).reshape(K, TB, D) + a2[...][None, :, :]
    u = _gelu(x).reshape(TE, D)
    u = _gelu(jnp.dot(u, w12[...], preferred_element_type=jnp.float32) + b12[...])
    msg2 = jnp.dot(u, w13[...], preferred_element_type=jnp.float32) + b13[...]
    ho = _ln(hee + msg2, ln3g[...], ln3b[...])
    he_out[...] = ho.reshape(1, K, TB, D)
    m3 = m_att[...][0][:, :, None]
    neigh[...] = (ho.reshape(K, TB, D) * m3).reshape(TE, D)


def _tc_c(he, g2, a2, m_att, p):
    return pl.pallas_call(
        _tcc_body,
        grid=(GRID,),
        in_specs=[
            pl.BlockSpec((1, K, TB, D), lambda i: (i // NPB, 0, i % NPB, 0)),
            pl.BlockSpec((TE, D), lambda i: (i, 0)),
            pl.BlockSpec((TB, D), lambda i: (i, 0)),
            pl.BlockSpec((1, K, TB), lambda i: (i, 0, 0)),
            _full((D, D)), _full((D, D)), _full((1, D)), _full((D, D)),
            _full((1, D)), _full((1, D)), _full((1, D)),
        ],
        out_specs=[
            pl.BlockSpec((1, K, TB, D), lambda i: (i // NPB, 0, i % NPB, 0)),
            pl.BlockSpec((TE, D), lambda i: (i, 0)),
        ],
        out_shape=[jax.ShapeDtypeStruct((B, K, N, D), jnp.float32),
                   jax.ShapeDtypeStruct((R, D), jnp.float32)],
    )(he, g2, a2, m_att,
      p['w11b'], p['w12'], p['b12'], p['w13'], p['b13'],
      p['ln3g'], p['ln3b'])


def _tcd_body(vals, cnts, h_mid, t1w, t1b, t2w, t2b,
              ln4g, ln4b, ln5g, ln5b, out):
    v = vals[...]
    c = cnts[...]
    c = jnp.where(c == 0.0, 1.0, c)
    ne = _ln(v / c, ln4g[...], ln4b[...])
    t = jax.nn.relu(jnp.dot(ne, t1w[...], preferred_element_type=jnp.float32)
                    + t1b[...])
    t = jnp.dot(t, t2w[...], preferred_element_type=jnp.float32) + t2b[...]
    ne = _ln(ne + t, ln5g[...], ln5b[...])
    out[...] = h_mid[...] + ne


def _tc_d(vals_p, cnts_p, h_mid, p):
    t = 512
    return pl.pallas_call(
        _tcd_body,
        grid=(BN // t,),
        in_specs=[
            pl.BlockSpec((t, D), lambda i: (i, 0)),
            pl.BlockSpec((t, D), lambda i: (i, 0)),
            pl.BlockSpec((t, D), lambda i: (i, 0)),
            _full((D, D)), _full((1, D)), _full((D, D)), _full((1, D)),
            _full((1, D)), _full((1, D)), _full((1, D)), _full((1, D)),
        ],
        out_specs=pl.BlockSpec((t, D), lambda i: (i, 0)),
        out_shape=jax.ShapeDtypeStruct((BN, D), jnp.float32),
    )(vals_p, cnts_p, h_mid,
      p['t1w'], p['t1b'], p['t2w'], p['t2b'],
      p['ln4g'], p['ln4b'], p['ln5g'], p['ln5b'])


NW = 32
EPT = R // NW
CH = 128
NCH = EPT // CH


def _sc_gather(table, idx2d):
    mesh = plsc.VectorSubcoreMesh(core_axis_name="c", subcore_axis_name="s",
                                  num_cores=2, num_subcores=16)

    @functools.partial(
        pl.kernel, mesh=mesh,
        out_type=jax.ShapeDtypeStruct((R, D), jnp.float32),
        scratch_types=[
            pltpu.VMEM((NCH, CH), jnp.int32),
            pltpu.VMEM((2, CH, D), jnp.float32),
            pltpu.SemaphoreType.DMA,
            pltpu.SemaphoreType.DMA,
        ],
    )
    def k(table_hbm, idx_hbm, out_hbm, idx_v, buf, gsem, wsem):
        wid = lax.axis_index("s") * 2 + lax.axis_index("c")
        base = wid * EPT
        pltpu.sync_copy(idx_hbm.at[pl.ds(wid * NCH, NCH)], idx_v)
        pltpu.async_copy(table_hbm.at[idx_v.at[0]], buf.at[0], gsem).wait()

        def body(j, _):
            slot = lax.rem(j, 2)
            nxt = lax.rem(j + 1, 2)
            jn = lax.min(j + 1, NCH - 1)
            h = pltpu.async_copy(table_hbm.at[idx_v.at[jn]], buf.at[nxt], gsem)
            pltpu.async_copy(
                buf.at[slot], out_hbm.at[pl.ds(base + j * CH, CH)], wsem).wait()
            h.wait()
            return 0

        lax.fori_loop(0, NCH, body, 0)

    return k(table, idx2d)


def _sc_scatter(neigh, idx2d, z128, ones128):
    mesh = plsc.VectorSubcoreMesh(core_axis_name="c", subcore_axis_name="s",
                                  num_cores=2, num_subcores=16)
    RPS = BN // 16
    EPT2 = R // 16
    NCH2 = EPT2 // CH

    @functools.partial(
        pl.kernel, mesh=mesh,
        out_type=[jax.ShapeDtypeStruct((BN, D), jnp.float32),
                  jax.ShapeDtypeStruct((BN, D), jnp.float32)],
        scratch_types=[
            pltpu.VMEM_SHARED((BN, D), jnp.float32),
            pltpu.VMEM((NCH2, CH), jnp.int32),
            pltpu.VMEM((2, CH, D), jnp.float32),
            pltpu.SemaphoreType.DMA,
        ],
    )
    def k(neigh_hbm, idx_hbm, z128_hbm, ones_hbm,
          vals_out, cnts_out,
          acc_sh, idx_v, rowbuf, lsem):
        cc = lax.axis_index("c")
        ss = lax.axis_index("s")
        pltpu.sync_copy(z128_hbm.at[pl.ds(ss * RPS, RPS)],
                        acc_sh.at[pl.ds(ss * RPS, RPS)])
        pltpu.sync_copy(idx_hbm.at[pl.ds(ss * NCH2, NCH2)], idx_v)

        @pl.when(cc == 1)
        def _():
            pltpu.sync_copy(ones_hbm, rowbuf.at[0])

        plsc.subcore_barrier()

        @pl.when(cc == 0)
        def _():
            pltpu.async_copy(neigh_hbm.at[pl.ds(ss * EPT2, CH)],
                             rowbuf.at[0], lsem).wait()

            def body(j, _):
                slot = lax.rem(j, 2)
                nxt = lax.rem(j + 1, 2)
                jn = lax.min(j + 1, NCH2 - 1)
                h = pltpu.async_copy(
                    neigh_hbm.at[pl.ds(ss * EPT2 + jn * CH, CH)],
                    rowbuf.at[nxt], lsem)
                pltpu.sync_copy(rowbuf.at[slot], acc_sh.at[idx_v.at[j]],
                                add=True)
                h.wait()
                return 0
            lax.fori_loop(0, NCH2, body, 0)

        @pl.when(cc == 1)
        def _():
            def body(j, _):
                pltpu.sync_copy(rowbuf.at[0], acc_sh.at[idx_v.at[j]],
                                add=True)
                return 0
            lax.fori_loop(0, NCH2, body, 0)

        plsc.subcore_barrier()
        for q in range(RPS // CH):
            r0 = ss * RPS + q * CH

            @pl.when(cc == 0)
            def _():
                pltpu.sync_copy(acc_sh.at[pl.ds(r0, CH)], rowbuf.at[0])
                pltpu.sync_copy(rowbuf.at[0], vals_out.at[pl.ds(r0, CH)])

            @pl.when(cc == 1)
            def _():
                pltpu.sync_copy(acc_sh.at[pl.ds(r0, CH)], rowbuf.at[0])
                pltpu.sync_copy(rowbuf.at[0], cnts_out.at[pl.ds(r0, CH)])

    return k(neigh, idx2d, z128, ones128)


def _gather_rows(table, idx2d):
    return jnp.take(table, idx2d.reshape(-1), axis=0)


def _scatter_mean_parts(neigh, idx2d):
    flat = idx2d.reshape(-1)
    vals = jnp.zeros((BN, D), jnp.float32).at[flat].add(neigh)
    cnts = jnp.zeros((BN,), jnp.float32).at[flat].add(1.0)
    return vals[None], cnts[None, :, None]


def kernel(h_V, h_E, mask_V, mask_attend, params, E_idx):
    p = dict(params)
    hv = h_V.reshape(BN, D)
    he_t = h_E.transpose(0, 2, 1, 3)
    m_att = (mask_attend.reshape(B, NPB, TB, K).transpose(0, 1, 3, 2)
             .reshape(GRID, K, TB))
    mask_v = mask_V.reshape(BN, 1)
    flat_idx = (E_idx.astype(jnp.int32)
                + (jnp.arange(B, dtype=jnp.int32) * N)[:, None, None])
    idx2d = (flat_idx.reshape(B, NPB, TB, K).transpose(0, 1, 3, 2)
             .reshape(R // 128, 128))

    p['w1a'], p['w1b'], p['w1c'] = p['w1'][:D], p['w1'][D:2 * D], p['w1'][2 * D:]
    p['w11a'], p['w11b'], p['w11c'] = (p['w11'][:D], p['w11'][D:2 * D],
                                       p['w11'][2 * D:])
    for k in ('b1', 'b2', 'b3', 'b11', 'b12', 'b13', 'bin', 'bout',
              't1b', 't2b', 'ln1g', 'ln1b', 'ln2g', 'ln2b', 'ln3g', 'ln3b',
              'ln4g', 'ln4b', 'ln5g', 'ln5b'):
        p[k] = p[k].reshape(1, -1)

    z128 = jnp.zeros((BN, D), jnp.float32)
    ones128 = jnp.ones((CH, D), jnp.float32)

    a1, hv3 = _tc_a(hv, p['w1a'], p['b1'], p['w1c'])
    g1 = _sc_gather(hv3, idx2d)
    h_mid, a2, hv3b = _tc_b(hv, a1, he_t, g1, m_att, mask_v, p)
    g2 = _sc_gather(hv3b, idx2d)
    he_out_t, neigh = _tc_c(he_t, g2, a2, m_att, p)
    vals, cnts = _sc_scatter(neigh, idx2d, z128, ones128)
    hv_out = _tc_d(vals, cnts, h_mid, p)
    return hv_out.reshape(B, N, D), he_out_t.transpose(0, 2, 1, 3)
